# Initial kernel scaffold; baseline (speedup 1.0000x reference)
import jax
import jax.numpy as jnp
from jax.experimental import pallas as pl


def kernel(x, gate_w, expert_w):
    raise NotImplementedError("write your pallas kernel here")


# speedup vs baseline: 2.0671x; 2.0671x over previous
import functools

import jax
import jax.numpy as jnp
from jax.experimental import pallas as pl
from jax.experimental.pallas import tpu as pltpu

E = 8
D = 1024
T = 4096
K = 2
N = T * K
BM = 256
NBLK = (N + E * (BM - 1) + BM - 1) // BM
NSLOT = NBLK * BM
TB = 512


def _routing_body(x_ref, gw_ref, idx_ref, w_ref):
    logits = jax.lax.dot_general(
        x_ref[...], gw_ref[...], (((1,), (1,)), ((), ())),
        preferred_element_type=jnp.float32)
    cols = jax.lax.broadcasted_iota(jnp.int32, (TB, E), 1)
    big = jnp.int32(E)
    m0 = jnp.max(logits, axis=1, keepdims=True)
    e0 = jnp.min(jnp.where(logits == m0, cols, big), axis=1, keepdims=True)
    neg = jnp.where(cols == e0, -jnp.inf, logits)
    m1 = jnp.max(neg, axis=1, keepdims=True)
    e1 = jnp.min(jnp.where(neg == m1, cols, big), axis=1, keepdims=True)
    t = jnp.exp(m1 - m0)
    w0 = 1.0 / (1.0 + t)
    w1 = 1.0 - w0
    idx_ref[...] = jnp.concatenate([e0, e1], axis=1)
    w_ref[...] = jnp.concatenate([w0, w1], axis=1)


def _routing(x, gate_w):
    return pl.pallas_call(
        _routing_body,
        grid=(T // TB,),
        in_specs=[
            pl.BlockSpec((TB, D), lambda i: (i, 0)),
            pl.BlockSpec((E, D), lambda i: (0, 0)),
        ],
        out_specs=[
            pl.BlockSpec((TB, K), lambda i: (i, 0)),
            pl.BlockSpec((TB, K), lambda i: (i, 0)),
        ],
        out_shape=[
            jax.ShapeDtypeStruct((T, K), jnp.int32),
            jax.ShapeDtypeStruct((T, K), jnp.float32),
        ],
    )(x, gate_w)


def _gmm_body(eob_ref, xs_ref, w_ref, out_ref):
    del eob_ref
    out_ref[...] = jax.lax.dot_general(
        xs_ref[...], w_ref[0], (((1,), (1,)), ((), ())),
        preferred_element_type=jnp.float32)


def _grouped_matmul(xs, expert_w, expert_of_block):
    grid_spec = pltpu.PrefetchScalarGridSpec(
        num_scalar_prefetch=1,
        grid=(NBLK,),
        in_specs=[
            pl.BlockSpec((BM, D), lambda b, eob: (b, 0)),
            pl.BlockSpec((1, D, D), lambda b, eob: (eob[b], 0, 0)),
        ],
        out_specs=pl.BlockSpec((BM, D), lambda b, eob: (b, 0)),
    )
    return pl.pallas_call(
        _gmm_body,
        grid_spec=grid_spec,
        out_shape=jax.ShapeDtypeStruct((NSLOT, D), jnp.float32),
    )(expert_of_block, xs, expert_w)


def kernel(x, gate_w, expert_w):
    idx, w = _routing(x, gate_w)

    flat_e = idx.reshape(-1)
    oh = (flat_e[:, None] == jnp.arange(E, dtype=jnp.int32)[None, :]
          ).astype(jnp.int32)
    rank = jnp.take_along_axis(jnp.cumsum(oh, axis=0) - oh,
                               flat_e[:, None], axis=1)[:, 0]
    count = jnp.sum(oh, axis=0)
    blocks_e = (count + BM - 1) // BM
    cb = jnp.concatenate([jnp.zeros((1,), jnp.int32),
                          jnp.cumsum(blocks_e).astype(jnp.int32)])
    padded_start = cb[:E] * BM
    dest = padded_start[flat_e] + rank
    src_tok = jnp.zeros((NSLOT,), jnp.int32).at[dest].set(
        jnp.arange(N, dtype=jnp.int32) // K)
    bids = jnp.arange(NBLK, dtype=jnp.int32)
    expert_of_block = jnp.sum(
        (bids[:, None] >= cb[None, 1:E]).astype(jnp.int32), axis=1)

    xs = x[src_tok]

    ys = _grouped_matmul(xs, expert_w, expert_of_block)

    dest2 = dest.reshape(T, K)
    out = (w[:, 0:1] * ys[dest2[:, 0]] + w[:, 1:2] * ys[dest2[:, 1]])
    return out
